# baseline (speedup 1.0000x reference)
import functools

import jax
import jax.numpy as jnp
from jax import lax
from jax.experimental import pallas as pl
from jax.experimental.pallas import tpu as pltpu
from jax.experimental.pallas import tpu_sc as plsc

N_TOKENS = 2048
DIM = 1024
HIDDEN = 1024
NUM_EXPERTS = 8
TOP_K = 2
EPAD = 128
N_SLOTS = N_TOKENS * TOP_K
BR = 512
NT = 16
NW = 32
TPW = N_TOKENS // NW
HALF = TPW // 2
DCH = 16
SUB = 128


def _router_body(x_ref, gwp_ref, xs1_ref, xs2_ref, d1_ref, d2_ref, meta_ref):
    x = x_ref[...]
    logits = lax.dot_general(x, gwp_ref[...], (((1,), (1,)), ((), ())),
                             preferred_element_type=jnp.float32)
    lane = lax.broadcasted_iota(jnp.int32, logits.shape, 1)
    scores = jax.nn.sigmoid(logits)
    scores = jnp.where(lane < NUM_EXPERTS, scores, -1.0)
    a1 = jnp.argmax(scores, axis=1)[:, None]
    oh1 = lane == a1
    a2 = jnp.argmax(jnp.where(oh1, -2.0, scores), axis=1)[:, None]
    oh2 = lane == a2
    s1 = jnp.sum(jnp.where(oh1, scores, 0.0), axis=1, keepdims=True)
    s2 = jnp.sum(jnp.where(oh2, scores, 0.0), axis=1, keepdims=True)
    xs1_ref[...] = x * s1
    xs2_ref[...] = x * s2

    c = oh1.astype(jnp.int32) + oh2.astype(jnp.int32)
    cnt = jnp.sum(c, axis=0, keepdims=True)

    def _excl_scan(v, axis):
        n = v.shape[axis]
        idx = lax.broadcasted_iota(jnp.int32, v.shape, axis)
        x = jnp.where(idx >= 1, jnp.roll(v, 1, axis), 0)
        s = 1
        while s < n:
            x = x + jnp.where(idx >= s, jnp.roll(x, s, axis), 0)
            s *= 2
        return x

    xc = _excl_scan(c, 0)
    offs_i = _excl_scan(cnt, 1)
    d1_ref[...] = jnp.sum(jnp.where(oh1, offs_i + xc, 0), axis=1,
                          keepdims=True)
    d2_ref[...] = jnp.sum(jnp.where(oh2, offs_i + xc, 0), axis=1,
                          keepdims=True)

    m0 = offs_i // BR
    mlast = (offs_i + cnt - 1) // BR
    ntiles = jnp.where(cnt > 0, mlast - m0 + 1, 0)
    start_tile = _excl_scan(ntiles, 1)
    end_tile = start_tile + ntiles
    total = jnp.sum(ntiles, axis=1, keepdims=True)

    ii = lax.broadcasted_iota(jnp.int32, (NT, EPAD), 0)
    lane2 = lax.broadcasted_iota(jnp.int32, (NT, EPAD), 1)
    valid_e = lane2 < NUM_EXPERTS
    e_i = jnp.sum((valid_e & (end_tile <= ii)).astype(jnp.int32),
                  axis=1, keepdims=True)
    e_i = jnp.minimum(e_i, NUM_EXPERTS - 1)
    ohe = lane2 == e_i
    pick = lambda arr: jnp.sum(jnp.where(ohe, arr, 0), axis=1, keepdims=True)
    i_col = lax.broadcasted_iota(jnp.int32, (NT, 1), 0)
    m_i = pick(m0) + i_col - pick(start_tile)
    g0_i = pick(offs_i)
    g1_i = g0_i + pick(cnt)
    active = i_col < total
    m_i = jnp.where(active, m_i, N_SLOTS // BR - 1)
    g0_i = jnp.where(active, g0_i, 0)
    g1_i = jnp.where(active, g1_i, 0)
    m_prev = jnp.concatenate([jnp.full((1, 1), -1, jnp.int32), m_i[:-1]], 0)
    firstv = (active & ((i_col == 0) | (m_i != m_prev))).astype(jnp.int32)
    meta_ref[...] = ((lane2 == 0) * e_i + (lane2 == 1) * m_i
                     + (lane2 == 2) * g0_i + (lane2 == 3) * g1_i
                     + (lane2 == 4) * firstv)


def _router(x2d, gwp):
    return pl.pallas_call(
        _router_body,
        out_shape=[
            jax.ShapeDtypeStruct((N_TOKENS, DIM), jnp.float32),
            jax.ShapeDtypeStruct((N_TOKENS, DIM), jnp.float32),
            jax.ShapeDtypeStruct((N_TOKENS, 1), jnp.int32),
            jax.ShapeDtypeStruct((N_TOKENS, 1), jnp.int32),
            jax.ShapeDtypeStruct((NT, EPAD), jnp.int32),
        ],
    )(x2d, gwp)


def _scatter_body(xs1, xs2, d1, d2, routed, idx_v, rows_v, sem):
    w = lax.axis_index("s") * 2 + lax.axis_index("c")
    base = w * TPW
    for src, didx in ((xs1, d1), (xs2, d2)):
        for hoff in (0, HALF):
            pltpu.sync_copy(didx.at[pl.ds(base + hoff, HALF)], idx_v)
            pltpu.sync_copy(src.at[pl.ds(base + hoff, HALF)], rows_v)
            pltpu.async_copy(rows_v, routed.at[idx_v], sem).wait()


def _scatter_rows(xs1, xs2, d1f, d2f):
    mesh = plsc.VectorSubcoreMesh(core_axis_name="c", subcore_axis_name="s")
    return pl.kernel(
        _scatter_body,
        out_type=jax.ShapeDtypeStruct((N_SLOTS, DIM), jnp.float32),
        mesh=mesh,
        scratch_types=[
            pltpu.VMEM((HALF,), jnp.int32),
            pltpu.VMEM((HALF, DIM), jnp.float32),
            pltpu.SemaphoreType.DMA,
        ],
    )(xs1, xs2, d1f, d2f)


def _group_body(meta, routed_ref, w1_ref, w3_ref, w2_ref, out_ref):
    i = pl.program_id(0)
    m = meta[i, 1]
    g0 = meta[i, 2]
    g1 = meta[i, 3]
    fv = meta[i, 4]

    @pl.when(fv == 1)
    def _zero():
        out_ref[...] = jnp.zeros_like(out_ref)

    lo = jnp.maximum(g0 - m * BR, 0) // SUB
    hi = -(-(jnp.minimum(g1 - m * BR, BR)) // SUB)
    hi = jnp.maximum(hi, lo)

    def _sub(sb, carry):
        r0 = sb * SUB
        gr = m * BR + r0 + lax.broadcasted_iota(jnp.int32, (SUB, 1), 0)
        mask = ((gr >= g0) & (gr < g1)).astype(jnp.float32)
        xm = (routed_ref[pl.ds(r0, SUB), :] * mask).astype(jnp.bfloat16)
        h1 = lax.dot_general(xm, w1_ref[0], (((1,), (1,)), ((), ())),
                             preferred_element_type=jnp.float32)
        h3 = lax.dot_general(xm, w3_ref[0], (((1,), (1,)), ((), ())),
                             preferred_element_type=jnp.float32)
        h = (jax.nn.silu(h1) * h3).astype(jnp.bfloat16)
        res = lax.dot_general(h, w2_ref[0], (((1,), (1,)), ((), ())),
                              preferred_element_type=jnp.float32)
        out_ref[pl.ds(r0, SUB), :] += res
        return carry

    lax.fori_loop(lo, hi, _sub, 0)


def _grouped_ffn(meta5, routed, w1, w3, w2):
    grid_spec = pltpu.PrefetchScalarGridSpec(
        num_scalar_prefetch=1,
        grid=(NT,),
        in_specs=[
            pl.BlockSpec((BR, DIM), lambda i, meta: (meta[i, 1], 0)),
            pl.BlockSpec((1, HIDDEN, DIM), lambda i, meta: (meta[i, 0], 0, 0)),
            pl.BlockSpec((1, HIDDEN, DIM), lambda i, meta: (meta[i, 0], 0, 0)),
            pl.BlockSpec((1, DIM, HIDDEN), lambda i, meta: (meta[i, 0], 0, 0)),
        ],
        out_specs=pl.BlockSpec((BR, DIM), lambda i, meta: (meta[i, 1], 0)),
    )
    return pl.pallas_call(
        _group_body,
        grid_spec=grid_spec,
        out_shape=jax.ShapeDtypeStruct((N_SLOTS, DIM), jnp.float32),
    )(meta5, routed, w1, w3, w2)


def _unsort_body(outs, d1, d2, out, i1_v, i2_v,
                 bA1, bA2, bB1, bB2, gsem, ssem):
    w = lax.axis_index("s") * 2 + lax.axis_index("c")
    base = w * TPW
    pltpu.sync_copy(d1.at[pl.ds(base, TPW)], i1_v)
    pltpu.sync_copy(d2.at[pl.ds(base, TPW)], i2_v)
    pairs = ((bA1, bA2), (bB1, bB2))
    nch = TPW // DCH
    gcps = {}
    scps = {}

    def fire(k):
        b1, b2 = pairs[k % 2]
        sl = pl.ds(k * DCH, DCH)
        gcps[k] = (
            pltpu.async_copy(outs.at[i1_v[sl]], b1, gsem),
            pltpu.async_copy(outs.at[i2_v[sl]], b2, gsem),
        )

    fire(0)
    for k in range(nch):
        if k + 1 < nch:
            if k - 1 >= 0:
                scps[k - 1].wait()
            fire(k + 1)
        c1, c2 = gcps[k]
        c1.wait()
        c2.wait()
        b1, b2 = pairs[k % 2]

        def _row_add(r, carry):
            for v in range(DIM // 16):
                sl2 = pl.ds(v * 16, 16)
                b1[r, sl2] = b1[r, sl2] + b2[r, sl2]
            return carry

        lax.fori_loop(0, DCH, _row_add, 0)
        scps[k] = pltpu.async_copy(b1, out.at[pl.ds(base + k * DCH, DCH)],
                                   ssem)
    scps[nch - 2].wait()
    scps[nch - 1].wait()


def _unsort_add(outs, d1f, d2f):
    mesh = plsc.VectorSubcoreMesh(core_axis_name="c", subcore_axis_name="s")
    return pl.kernel(
        _unsort_body,
        out_type=jax.ShapeDtypeStruct((N_TOKENS, DIM), jnp.float32),
        mesh=mesh,
        scratch_types=[
            pltpu.VMEM((TPW,), jnp.int32),
            pltpu.VMEM((TPW,), jnp.int32),
            pltpu.VMEM((DCH, DIM), jnp.float32),
            pltpu.VMEM((DCH, DIM), jnp.float32),
            pltpu.VMEM((DCH, DIM), jnp.float32),
            pltpu.VMEM((DCH, DIM), jnp.float32),
            pltpu.SemaphoreType.DMA,
            pltpu.SemaphoreType.DMA,
        ],
    )(outs, d1f, d2f)


@jax.jit
def _moe(x2d, gwp, w1, w2, w3):
    xs1, xs2, d1, d2, meta = _router(x2d, gwp)
    d1f = d1.reshape(N_TOKENS)
    d2f = d2.reshape(N_TOKENS)
    routed = _scatter_rows(xs1, xs2, d1f, d2f)
    outs = _grouped_ffn(meta[:, :5], routed,
                        w1.astype(jnp.bfloat16), w3.astype(jnp.bfloat16),
                        w2.astype(jnp.bfloat16))
    return _unsort_add(outs, d1f, d2f)


def kernel(x, gate_w, w1, w2, w3):
    orig_shape = x.shape
    x2d = x.reshape(-1, orig_shape[-1])
    gwp = jnp.zeros((EPAD, DIM), jnp.float32).at[:NUM_EXPERTS].set(gate_w)
    out = _moe(x2d, gwp, w1, w2, w3)
    return out.reshape(orig_shape)


# speedup vs baseline: 1.4979x; 1.4979x over previous
import functools

import jax
import jax.numpy as jnp
from jax import lax
from jax.experimental import pallas as pl
from jax.experimental.pallas import tpu as pltpu
from jax.experimental.pallas import tpu_sc as plsc

N_TOKENS = 2048
DIM = 1024
HIDDEN = 1024
NUM_EXPERTS = 8
TOP_K = 2
EPAD = 128
N_SLOTS = N_TOKENS * TOP_K
BR = 512
NT = 16
NW = 32
TPW = N_TOKENS // NW
HALF = TPW // 2
DCH = 16
SUB = 128


def _router_body(x_ref, gwp_ref, xs1_ref, xs2_ref, d1_ref, d2_ref, meta_ref):
    x = x_ref[...]
    logits = lax.dot_general(x, gwp_ref[...], (((1,), (1,)), ((), ())),
                             preferred_element_type=jnp.float32)
    lane = lax.broadcasted_iota(jnp.int32, logits.shape, 1)
    scores = jax.nn.sigmoid(logits)
    scores = jnp.where(lane < NUM_EXPERTS, scores, -1.0)
    a1 = jnp.argmax(scores, axis=1)[:, None]
    oh1 = lane == a1
    a2 = jnp.argmax(jnp.where(oh1, -2.0, scores), axis=1)[:, None]
    oh2 = lane == a2
    s1 = jnp.sum(jnp.where(oh1, scores, 0.0), axis=1, keepdims=True)
    s2 = jnp.sum(jnp.where(oh2, scores, 0.0), axis=1, keepdims=True)
    xs1_ref[...] = x * s1
    xs2_ref[...] = x * s2

    c = oh1.astype(jnp.int32) + oh2.astype(jnp.int32)
    cnt = jnp.sum(c, axis=0, keepdims=True)

    def _excl_scan(v, axis):
        n = v.shape[axis]
        idx = lax.broadcasted_iota(jnp.int32, v.shape, axis)
        x = jnp.where(idx >= 1, jnp.roll(v, 1, axis), 0)
        s = 1
        while s < n:
            x = x + jnp.where(idx >= s, jnp.roll(x, s, axis), 0)
            s *= 2
        return x

    xc = _excl_scan(c, 0)
    offs_i = _excl_scan(cnt, 1)
    d1_ref[...] = jnp.sum(jnp.where(oh1, offs_i + xc, 0), axis=1,
                          keepdims=True)
    d2_ref[...] = jnp.sum(jnp.where(oh2, offs_i + xc, 0), axis=1,
                          keepdims=True)

    m0 = offs_i // BR
    mlast = (offs_i + cnt - 1) // BR
    ntiles = jnp.where(cnt > 0, mlast - m0 + 1, 0)
    start_tile = _excl_scan(ntiles, 1)
    end_tile = start_tile + ntiles
    total = jnp.sum(ntiles, axis=1, keepdims=True)

    ii = lax.broadcasted_iota(jnp.int32, (NT, EPAD), 0)
    lane2 = lax.broadcasted_iota(jnp.int32, (NT, EPAD), 1)
    valid_e = lane2 < NUM_EXPERTS
    e_i = jnp.sum((valid_e & (end_tile <= ii)).astype(jnp.int32),
                  axis=1, keepdims=True)
    e_i = jnp.minimum(e_i, NUM_EXPERTS - 1)
    ohe = lane2 == e_i
    pick = lambda arr: jnp.sum(jnp.where(ohe, arr, 0), axis=1, keepdims=True)
    i_col = lax.broadcasted_iota(jnp.int32, (NT, 1), 0)
    m_i = pick(m0) + i_col - pick(start_tile)
    g0_i = pick(offs_i)
    g1_i = g0_i + pick(cnt)
    active = i_col < total
    m_i = jnp.where(active, m_i, N_SLOTS // BR - 1)
    g0_i = jnp.where(active, g0_i, 0)
    g1_i = jnp.where(active, g1_i, 0)
    m_prev = jnp.concatenate([jnp.full((1, 1), -1, jnp.int32), m_i[:-1]], 0)
    firstv = (active & ((i_col == 0) | (m_i != m_prev))).astype(jnp.int32)
    meta_ref[...] = ((lane2 == 0) * e_i + (lane2 == 1) * m_i
                     + (lane2 == 2) * g0_i + (lane2 == 3) * g1_i
                     + (lane2 == 4) * firstv)


def _router(x2d, gwp):
    return pl.pallas_call(
        _router_body,
        out_shape=[
            jax.ShapeDtypeStruct((N_TOKENS, DIM), jnp.float32),
            jax.ShapeDtypeStruct((N_TOKENS, DIM), jnp.float32),
            jax.ShapeDtypeStruct((N_TOKENS, 1), jnp.int32),
            jax.ShapeDtypeStruct((N_TOKENS, 1), jnp.int32),
            jax.ShapeDtypeStruct((NT, EPAD), jnp.int32),
        ],
    )(x2d, gwp)


def _scatter_body(xs1, xs2, d1, d2, routed, i1_v, i2_v,
                  b0, b1, b2, b3, lsem, ssem):
    w = lax.axis_index("s") * 2 + lax.axis_index("c")
    base = w * TPW
    pltpu.sync_copy(d1.at[pl.ds(base, TPW)], i1_v)
    pltpu.sync_copy(d2.at[pl.ds(base, TPW)], i2_v)
    bufs = (b0, b1, b2, b3)
    units = [(xs1 if j % 2 == 0 else xs2,
              i1_v if j % 2 == 0 else i2_v,
              (j // 2) * DCH) for j in range(2 * (TPW // DCH))]
    nu = len(units)
    lcps = {}
    scps = {}

    def load(j):
        src, _, off = units[j]
        lcps[j] = pltpu.async_copy(src.at[pl.ds(base + off, DCH)],
                                   bufs[j % 4], lsem)

    for j in range(4):
        load(j)
    for j in range(nu):
        _, iv_ref, off = units[j]
        lcps[j].wait()
        scps[j] = pltpu.async_copy(bufs[j % 4],
                                   routed.at[iv_ref[pl.ds(off, DCH)]], ssem)
        if j + 4 < nu:
            scps[j].wait()
            load(j + 4)
    for j in range(nu - 4, nu):
        scps[j].wait()


def _scatter_rows(xs1, xs2, d1f, d2f):
    mesh = plsc.VectorSubcoreMesh(core_axis_name="c", subcore_axis_name="s")
    return pl.kernel(
        _scatter_body,
        out_type=jax.ShapeDtypeStruct((N_SLOTS, DIM), jnp.float32),
        mesh=mesh,
        scratch_types=[
            pltpu.VMEM((TPW,), jnp.int32),
            pltpu.VMEM((TPW,), jnp.int32),
            pltpu.VMEM((DCH, DIM), jnp.float32),
            pltpu.VMEM((DCH, DIM), jnp.float32),
            pltpu.VMEM((DCH, DIM), jnp.float32),
            pltpu.VMEM((DCH, DIM), jnp.float32),
            pltpu.SemaphoreType.DMA,
            pltpu.SemaphoreType.DMA,
        ],
    )(xs1, xs2, d1f, d2f)


def _group_body(meta, routed_ref, w1_ref, w3_ref, w2_ref, out_ref):
    i = pl.program_id(0)
    m = meta[i, 1]
    g0 = meta[i, 2]
    g1 = meta[i, 3]
    fv = meta[i, 4]

    @pl.when(g1 > g0)
    def _compute():
        gr = m * BR + lax.broadcasted_iota(jnp.int32, (BR, 1), 0)
        mask = ((gr >= g0) & (gr < g1)).astype(jnp.float32)
        xm = routed_ref[...] * mask
        h1 = lax.dot_general(xm, w1_ref[0], (((1,), (1,)), ((), ())),
                             preferred_element_type=jnp.float32)
        h3 = lax.dot_general(xm, w3_ref[0], (((1,), (1,)), ((), ())),
                             preferred_element_type=jnp.float32)
        h = jax.nn.silu(h1) * h3
        res = lax.dot_general(h, w2_ref[0], (((1,), (1,)), ((), ())),
                              preferred_element_type=jnp.float32)

        @pl.when(fv == 1)
        def _init():
            out_ref[...] = res

        @pl.when(fv == 0)
        def _acc():
            out_ref[...] += res


def _grouped_ffn(meta5, routed, w1, w3, w2):
    grid_spec = pltpu.PrefetchScalarGridSpec(
        num_scalar_prefetch=1,
        grid=(NT,),
        in_specs=[
            pl.BlockSpec((BR, DIM), lambda i, meta: (meta[i, 1], 0)),
            pl.BlockSpec((1, HIDDEN, DIM), lambda i, meta: (meta[i, 0], 0, 0)),
            pl.BlockSpec((1, HIDDEN, DIM), lambda i, meta: (meta[i, 0], 0, 0)),
            pl.BlockSpec((1, DIM, HIDDEN), lambda i, meta: (meta[i, 0], 0, 0)),
        ],
        out_specs=pl.BlockSpec((BR, DIM), lambda i, meta: (meta[i, 1], 0)),
    )
    return pl.pallas_call(
        _group_body,
        grid_spec=grid_spec,
        out_shape=jax.ShapeDtypeStruct((N_SLOTS, DIM), jnp.float32),
    )(meta5, routed, w1, w3, w2)


def _unsort_body(outs, d1, d2, out, i1_v, i2_v,
                 bA1, bA2, bB1, bB2, gsem, ssem):
    w = lax.axis_index("s") * 2 + lax.axis_index("c")
    base = w * TPW
    pltpu.sync_copy(d1.at[pl.ds(base, TPW)], i1_v)
    pltpu.sync_copy(d2.at[pl.ds(base, TPW)], i2_v)
    pairs = ((bA1, bA2), (bB1, bB2))
    nch = TPW // DCH
    gcps = {}
    scps = {}

    def fire(k):
        b1, b2 = pairs[k % 2]
        sl = pl.ds(k * DCH, DCH)
        gcps[k] = (
            pltpu.async_copy(outs.at[i1_v[sl]], b1, gsem),
            pltpu.async_copy(outs.at[i2_v[sl]], b2, gsem),
        )

    fire(0)
    for k in range(nch):
        if k + 1 < nch:
            if k - 1 >= 0:
                scps[k - 1].wait()
            fire(k + 1)
        c1, c2 = gcps[k]
        c1.wait()
        c2.wait()
        b1, b2 = pairs[k % 2]

        def _row_add(r, carry):
            for v in range(DIM // 16):
                sl2 = pl.ds(v * 16, 16)
                b1[r, sl2] = b1[r, sl2] + b2[r, sl2]
            return carry

        lax.fori_loop(0, DCH, _row_add, 0)
        scps[k] = pltpu.async_copy(b1, out.at[pl.ds(base + k * DCH, DCH)],
                                   ssem)
    scps[nch - 2].wait()
    scps[nch - 1].wait()


def _unsort_add(outs, d1f, d2f):
    mesh = plsc.VectorSubcoreMesh(core_axis_name="c", subcore_axis_name="s")
    return pl.kernel(
        _unsort_body,
        out_type=jax.ShapeDtypeStruct((N_TOKENS, DIM), jnp.float32),
        mesh=mesh,
        scratch_types=[
            pltpu.VMEM((TPW,), jnp.int32),
            pltpu.VMEM((TPW,), jnp.int32),
            pltpu.VMEM((DCH, DIM), jnp.float32),
            pltpu.VMEM((DCH, DIM), jnp.float32),
            pltpu.VMEM((DCH, DIM), jnp.float32),
            pltpu.VMEM((DCH, DIM), jnp.float32),
            pltpu.SemaphoreType.DMA,
            pltpu.SemaphoreType.DMA,
        ],
    )(outs, d1f, d2f)


@jax.jit
def _moe(x2d, gwp, w1, w2, w3):
    xs1, xs2, d1, d2, meta = _router(x2d, gwp)
    d1f = d1.reshape(N_TOKENS)
    d2f = d2.reshape(N_TOKENS)
    routed = _scatter_rows(xs1, xs2, d1f, d2f)
    outs = _grouped_ffn(meta[:, :5], routed, w1, w3, w2)
    return _unsort_add(outs, d1f, d2f)


def kernel(x, gate_w, w1, w2, w3):
    orig_shape = x.shape
    x2d = x.reshape(-1, orig_shape[-1])
    gwp = jnp.zeros((EPAD, DIM), jnp.float32).at[:NUM_EXPERTS].set(gate_w)
    out = _moe(x2d, gwp, w1, w2, w3)
    return out.reshape(orig_shape)
